# baseline (speedup 1.0000x reference)
import functools

import jax
import jax.numpy as jnp
import numpy as np
from jax import lax
from jax.experimental import pallas as pl
from jax.experimental.pallas import tpu as pltpu
from jax.experimental.pallas import tpu_sc as plsc

_P = 2
_K = 24
_C = 64
_W = 128
_WP = _W + 2 * _P
_N_TB = 2 * _P * _WP
_N_LR = _W * 2 * _P
_N_HALO = _N_TB + _N_LR
_N_PAD = 1152
_ROWS = _N_PAD // 128
_NW = 32
_SLICES = _K * _C
_PER_W = _SLICES // _NW


_SLICE_WORDS = _WP * _WP
_BUF_WORDS = _SLICE_WORDS + 8


def _halo_dest_np():
    dest = np.zeros((_N_PAD,), dtype=np.int32)
    s = np.arange(_N_TB)
    i, j = s // _WP, s % _WP
    rows = np.where(i < _P, i, _W + i)
    dest[:_N_TB] = rows * _WP + j
    t = np.arange(_N_LR)
    r, c = t // (2 * _P), t % (2 * _P)
    cols = np.where(c < _P, c, _W + c)
    dest[_N_TB:_N_HALO] = (_P + r) * _WP + cols
    dest[_N_HALO:] = _SLICE_WORDS + (np.arange(_N_PAD - _N_HALO) % 8)
    return dest.reshape(_ROWS, 128)


_DEST = _halo_dest_np()


def _halo_indices(to_process, batch_size):
    c, w, p = _C, _W, _P
    wp = _WP
    t = (2.0 * (jnp.arange(wp, dtype=jnp.float32) - p) + 1.0 - w) / w
    u = jnp.broadcast_to(t[None, :], (wp, wp))
    v = jnp.broadcast_to(t[:, None], (wp, wp))
    one = jnp.ones((wp, wp), dtype=jnp.float32)
    dirs = jnp.stack([
        jnp.stack([one, -v, -u], axis=-1),
        jnp.stack([-one, -v, u], axis=-1),
        jnp.stack([u, one, v], axis=-1),
        jnp.stack([u, -one, -v], axis=-1),
        jnp.stack([u, -v, one], axis=-1),
        jnp.stack([-u, -v, -one], axis=-1),
    ], axis=0)
    x, y, z = dirs[..., 0], dirs[..., 1], dirs[..., 2]
    ax, ay, az = jnp.abs(x), jnp.abs(y), jnp.abs(z)
    is_x = (ax >= ay) & (ax >= az)
    is_y = jnp.logical_and(~is_x, ay >= az)
    face = jnp.where(is_x, jnp.where(x > 0, 0, 1),
           jnp.where(is_y, jnp.where(y > 0, 2, 3),
                     jnp.where(z > 0, 4, 5)))
    a = jnp.maximum(jnp.maximum(ax, ay), az)
    uc = jnp.stack([-z, z, x, x, x, -x], axis=0) / a
    vc = jnp.stack([-y, -y, z, -z, -y, -y], axis=0) / a
    u2 = jnp.take_along_axis(uc, face[None], axis=0)[0]
    v2 = jnp.take_along_axis(vc, face[None], axis=0)[0]
    jj = jnp.clip(jnp.floor((u2 + 1.0) * 0.5 * w), 0, w - 1).astype(jnp.int32)
    ii = jnp.clip(jnp.floor((v2 + 1.0) * 0.5 * w), 0, w - 1).astype(jnp.int32)
    pix = ii * w + jj
    face = face.astype(jnp.int32)
    pix_tb = jnp.concatenate([pix[:, :p, :], pix[:, wp - p:, :]], axis=1)
    pix_lr = jnp.concatenate([pix[:, p:wp - p, :p], pix[:, p:wp - p, wp - p:]], axis=2)
    f_tb = jnp.concatenate([face[:, :p, :], face[:, wp - p:, :]], axis=1)
    f_lr = jnp.concatenate([face[:, p:wp - p, :p], face[:, p:wp - p, wp - p:]], axis=2)

    n_faces = to_process.shape[0]
    k = n_faces
    bs = n_faces // 6
    bs_delta = jnp.asarray(batch_size, dtype=jnp.int32) - bs
    inv = jnp.full((n_faces,), -1, dtype=jnp.int32)
    inv = inv.at[to_process].set(jnp.arange(k, dtype=jnp.int32))
    boff = 6 * (jnp.arange(bs, dtype=jnp.int32) + bs_delta)[:, None, None]
    ftb = (f_tb[None] + boff[:, :, None]).reshape(n_faces, 2 * p, wp)[to_process]
    ftb = inv[ftb]
    flr = (f_lr[None] + boff[:, :, None]).reshape(n_faces, w, 2 * p)[to_process]
    flr = inv[flr]
    ptb = pix_tb[to_process % 6]
    plr = pix_lr[to_process % 6]
    base_tb = ftb * (c * w * w) + ptb
    base_lr = flr * (c * w * w) + plr
    m_tb = (ftb >= 0)
    m_lr = (flr >= 0)
    base = jnp.concatenate([
        jnp.where(m_tb, base_tb, 0).reshape(n_faces, _N_TB),
        jnp.where(m_lr, base_lr, 0).reshape(n_faces, _N_LR),
        jnp.zeros((n_faces, _N_PAD - _N_HALO), jnp.int32),
    ], axis=1)
    mask = jnp.concatenate([
        m_tb.reshape(n_faces, _N_TB).astype(jnp.float32),
        m_lr.reshape(n_faces, _N_LR).astype(jnp.float32),
        jnp.zeros((n_faces, _N_PAD - _N_HALO), jnp.float32),
    ], axis=1)
    return base, mask


def _sc_body(idx_hbm, mask_hbm, cube_flat, dest_hbm, out_hbm,
             idxv, hv, mv, dfv, cbuf, buf, gsem):
    cid = lax.axis_index("c")
    sid = lax.axis_index("s")
    wid = sid * 2 + cid
    pltpu.sync_copy(dest_hbm, dfv)
    viota = lax.iota(jnp.int32, 16)

    def slice_body(i, carry):
        s = wid * _PER_W + i
        k = s // _C
        ch = s - k * _C
        src_off = pl.multiple_of(s * (_W * _W), 8)
        out_off = pl.multiple_of(s * _SLICE_WORDS, 8)
        pltpu.sync_copy(idx_hbm.at[k, ch], idxv)
        pltpu.sync_copy(mask_hbm.at[k], mv)
        pltpu.sync_copy(cube_flat.at[pl.ds(src_off, _W * _W)], cbuf)
        gathers = [pltpu.async_copy(cube_flat.at[idxv.at[r]], hv.at[r], gsem)
                   for r in range(_ROWS)]
        for gcp in gathers:
            gcp.wait()
        for j in range(_W * _W // 16):
            r, c0 = j // 8, (j % 8) * 16
            dbase = (_P + r) * _WP + _P + c0
            plsc.store_scatter(buf, [viota + dbase], cbuf[pl.ds(j * 16, 16)])
        for g in range(_ROWS * 8):
            r, c0 = g // 8, (g % 8) * 16
            val = hv[r, pl.ds(c0, 16)] * mv[r, pl.ds(c0, 16)]
            plsc.store_scatter(buf, [dfv[r, pl.ds(c0, 16)]], val)
        pltpu.sync_copy(buf.at[pl.ds(0, _SLICE_WORDS)],
                        out_hbm.at[pl.ds(out_off, _SLICE_WORDS)])
        return carry

    lax.fori_loop(0, _PER_W, slice_body, 0)


@functools.partial(jax.jit, static_argnums=())
def _sc_pad(idx_full, mask, cube, dest):
    mesh = plsc.VectorSubcoreMesh(core_axis_name="c", subcore_axis_name="s",
                                  num_cores=2, num_subcores=16)
    f = pl.kernel(
        _sc_body,
        out_type=jax.ShapeDtypeStruct((_K * _C * _SLICE_WORDS,), jnp.float32),
        mesh=mesh,
        scratch_types=[
            pltpu.VMEM((_ROWS, 128), jnp.int32),
            pltpu.VMEM((_ROWS, 128), jnp.float32),
            pltpu.VMEM((_ROWS, 128), jnp.float32),
            pltpu.VMEM((_ROWS, 128), jnp.int32),
            pltpu.VMEM((_W * _W,), jnp.float32),
            pltpu.VMEM((_BUF_WORDS,), jnp.float32),
            pltpu.SemaphoreType.DMA,
        ],
        compiler_params=pltpu.CompilerParams(use_tc_tiling_on_sc=False,
                                             needs_layout_passes=False),
    )
    return f(idx_full, mask, cube.reshape(-1), dest)


def kernel(cube, to_process, batch_size):
    base, mask = _halo_indices(to_process, batch_size)
    ch_off = (jnp.arange(_C, dtype=jnp.int32) * (_W * _W))[None, :, None]
    idx_full = (base[:, None, :] + ch_off).reshape(_K, _C, _ROWS, 128)
    mask = mask.reshape(_K, _ROWS, 128)
    out = _sc_pad(idx_full, mask, cube, jnp.asarray(_DEST))
    return out.reshape(_K, _C, _WP, _WP)


# speedup vs baseline: 1.1203x; 1.1203x over previous
import functools

import jax
import jax.numpy as jnp
import numpy as np
from jax import lax
from jax.experimental import pallas as pl
from jax.experimental.pallas import tpu as pltpu
from jax.experimental.pallas import tpu_sc as plsc

_P = 2
_K = 24
_C = 64
_W = 128
_WP = _W + 2 * _P
_N_TB = 2 * _P * _WP
_N_LR = _W * 2 * _P
_N_HALO = _N_TB + _N_LR
_N_PAD = 1152
_ROWS = _N_PAD // 128
_NW = 32
_SLICES = _K * _C
_PER_W = _SLICES // _NW


_SLICE_WORDS = _WP * _WP
_BUF_WORDS = _SLICE_WORDS + 8


def _halo_dest_np():
    dest = np.zeros((_N_PAD,), dtype=np.int32)
    s = np.arange(_N_TB)
    i, j = s // _WP, s % _WP
    rows = np.where(i < _P, i, _W + i)
    dest[:_N_TB] = rows * _WP + j
    t = np.arange(_N_LR)
    r, c = t // (2 * _P), t % (2 * _P)
    cols = np.where(c < _P, c, _W + c)
    dest[_N_TB:_N_HALO] = (_P + r) * _WP + cols
    dest[_N_HALO:] = _SLICE_WORDS + (np.arange(_N_PAD - _N_HALO) % 8)
    return dest.reshape(_ROWS, 128)


_DEST = _halo_dest_np()


def _halo_indices(to_process, batch_size):
    c, w, p = _C, _W, _P
    wp = _WP
    t = (2.0 * (jnp.arange(wp, dtype=jnp.float32) - p) + 1.0 - w) / w
    u = jnp.broadcast_to(t[None, :], (wp, wp))
    v = jnp.broadcast_to(t[:, None], (wp, wp))
    one = jnp.ones((wp, wp), dtype=jnp.float32)
    dirs = jnp.stack([
        jnp.stack([one, -v, -u], axis=-1),
        jnp.stack([-one, -v, u], axis=-1),
        jnp.stack([u, one, v], axis=-1),
        jnp.stack([u, -one, -v], axis=-1),
        jnp.stack([u, -v, one], axis=-1),
        jnp.stack([-u, -v, -one], axis=-1),
    ], axis=0)
    x, y, z = dirs[..., 0], dirs[..., 1], dirs[..., 2]
    ax, ay, az = jnp.abs(x), jnp.abs(y), jnp.abs(z)
    is_x = (ax >= ay) & (ax >= az)
    is_y = jnp.logical_and(~is_x, ay >= az)
    face = jnp.where(is_x, jnp.where(x > 0, 0, 1),
           jnp.where(is_y, jnp.where(y > 0, 2, 3),
                     jnp.where(z > 0, 4, 5)))
    a = jnp.maximum(jnp.maximum(ax, ay), az)
    uc = jnp.stack([-z, z, x, x, x, -x], axis=0) / a
    vc = jnp.stack([-y, -y, z, -z, -y, -y], axis=0) / a
    u2 = jnp.take_along_axis(uc, face[None], axis=0)[0]
    v2 = jnp.take_along_axis(vc, face[None], axis=0)[0]
    jj = jnp.clip(jnp.floor((u2 + 1.0) * 0.5 * w), 0, w - 1).astype(jnp.int32)
    ii = jnp.clip(jnp.floor((v2 + 1.0) * 0.5 * w), 0, w - 1).astype(jnp.int32)
    pix = ii * w + jj
    face = face.astype(jnp.int32)
    pix_tb = jnp.concatenate([pix[:, :p, :], pix[:, wp - p:, :]], axis=1)
    pix_lr = jnp.concatenate([pix[:, p:wp - p, :p], pix[:, p:wp - p, wp - p:]], axis=2)
    f_tb = jnp.concatenate([face[:, :p, :], face[:, wp - p:, :]], axis=1)
    f_lr = jnp.concatenate([face[:, p:wp - p, :p], face[:, p:wp - p, wp - p:]], axis=2)

    n_faces = to_process.shape[0]
    k = n_faces
    bs = n_faces // 6
    bs_delta = jnp.asarray(batch_size, dtype=jnp.int32) - bs
    inv = jnp.full((n_faces,), -1, dtype=jnp.int32)
    inv = inv.at[to_process].set(jnp.arange(k, dtype=jnp.int32))
    boff = 6 * (jnp.arange(bs, dtype=jnp.int32) + bs_delta)[:, None, None]
    ftb = (f_tb[None] + boff[:, :, None]).reshape(n_faces, 2 * p, wp)[to_process]
    ftb = inv[ftb]
    flr = (f_lr[None] + boff[:, :, None]).reshape(n_faces, w, 2 * p)[to_process]
    flr = inv[flr]
    ptb = pix_tb[to_process % 6]
    plr = pix_lr[to_process % 6]
    base_tb = ftb * (c * w * w) + ptb
    base_lr = flr * (c * w * w) + plr
    m_tb = (ftb >= 0)
    m_lr = (flr >= 0)
    base = jnp.concatenate([
        jnp.where(m_tb, base_tb, 0).reshape(n_faces, _N_TB),
        jnp.where(m_lr, base_lr, 0).reshape(n_faces, _N_LR),
        jnp.zeros((n_faces, _N_PAD - _N_HALO), jnp.int32),
    ], axis=1)
    mask = jnp.concatenate([
        m_tb.reshape(n_faces, _N_TB).astype(jnp.float32),
        m_lr.reshape(n_faces, _N_LR).astype(jnp.float32),
        jnp.zeros((n_faces, _N_PAD - _N_HALO), jnp.float32),
    ], axis=1)
    return base, mask


def _sc_body(idx_hbm, mask_hbm, cube_flat, dest_hbm, out_hbm,
             idxv0, idxv1, hv0, hv1, mv0, mv1, cbuf0, cbuf1, buf0, buf1, dfv,
             isem0, isem1, lsem0, lsem1, gsem0, gsem1, osem0, osem1):
    idxv = (idxv0, idxv1)
    hv = (hv0, hv1)
    mv = (mv0, mv1)
    cbuf = (cbuf0, cbuf1)
    buf = (buf0, buf1)
    isem = (isem0, isem1)
    lsem = (lsem0, lsem1)
    gsem = (gsem0, gsem1)
    osem = (osem0, osem1)
    cid = lax.axis_index("c")
    sid = lax.axis_index("s")
    wid = sid * 2 + cid
    base_s = wid * _PER_W
    pltpu.sync_copy(dest_hbm, dfv)
    viota = lax.iota(jnp.int32, 16)
    n_iter = _PER_W // 2

    def kch(s):
        s = jnp.minimum(s, _SLICES - 1)
        k = s // _C
        return s, k, s - k * _C

    def start(slot, s):
        s, k, ch = kch(s)
        src_off = pl.multiple_of(s * (_W * _W), 8)
        pltpu.async_copy(idx_hbm.at[k, ch], idxv[slot], isem[slot])
        pltpu.async_copy(mask_hbm.at[k], mv[slot], lsem[slot])
        pltpu.async_copy(cube_flat.at[pl.ds(src_off, _W * _W)],
                         cbuf[slot], lsem[slot])

    def mid(slot):
        pltpu.make_async_copy(idx_hbm.at[0, 0], idxv[slot], isem[slot]).wait()
        for r in range(_ROWS):
            pltpu.async_copy(cube_flat.at[idxv[slot].at[r]],
                             hv[slot].at[r], gsem[slot])

    def finish(slot, s, t):
        s, k, ch = kch(s)
        out_off = pl.multiple_of(s * _SLICE_WORDS, 8)
        pltpu.make_async_copy(mask_hbm.at[0], mv[slot], lsem[slot]).wait()
        pltpu.make_async_copy(cube_flat.at[pl.ds(0, _W * _W)],
                              cbuf[slot], lsem[slot]).wait()
        for r in range(_ROWS):
            pltpu.make_async_copy(cube_flat.at[pl.ds(0, 128)],
                                  hv[slot].at[r], gsem[slot]).wait()

        @pl.when(t > 0)
        def _wait_prev_write():
            pltpu.make_async_copy(buf[slot].at[pl.ds(0, _SLICE_WORDS)],
                                  out_hbm.at[pl.ds(0, _SLICE_WORDS)],
                                  osem[slot]).wait()

        for j in range(_W * _W // 16):
            r, c0 = j // 8, (j % 8) * 16
            dbase = (_P + r) * _WP + _P + c0
            plsc.store_scatter(buf[slot], [viota + dbase],
                               cbuf[slot][pl.ds(j * 16, 16)])
        for g in range(_ROWS * 8):
            r, c0 = g // 8, (g % 8) * 16
            val = hv[slot][r, pl.ds(c0, 16)] * mv[slot][r, pl.ds(c0, 16)]
            plsc.store_scatter(buf[slot], [dfv[r, pl.ds(c0, 16)]], val)
        pltpu.async_copy(buf[slot].at[pl.ds(0, _SLICE_WORDS)],
                         out_hbm.at[pl.ds(out_off, _SLICE_WORDS)], osem[slot])

    start(0, base_s)
    mid(0)

    def body(t, carry):
        s0 = base_s + 2 * t
        start(1, s0 + 1)
        finish(0, s0, t)
        mid(1)

        @pl.when(t < n_iter - 1)
        def _next():
            start(0, s0 + 2)
            mid(0)

        finish(1, s0 + 1, t)
        return carry

    lax.fori_loop(0, n_iter, body, 0)
    pltpu.make_async_copy(buf0.at[pl.ds(0, _SLICE_WORDS)],
                          out_hbm.at[pl.ds(0, _SLICE_WORDS)], osem0).wait()
    pltpu.make_async_copy(buf1.at[pl.ds(0, _SLICE_WORDS)],
                          out_hbm.at[pl.ds(0, _SLICE_WORDS)], osem1).wait()


@functools.partial(jax.jit, static_argnums=())
def _sc_pad(idx_full, mask, cube, dest):
    mesh = plsc.VectorSubcoreMesh(core_axis_name="c", subcore_axis_name="s",
                                  num_cores=2, num_subcores=16)
    f = pl.kernel(
        _sc_body,
        out_type=jax.ShapeDtypeStruct((_K * _C * _SLICE_WORDS,), jnp.float32),
        mesh=mesh,
        scratch_types=[
            pltpu.VMEM((_ROWS, 128), jnp.int32),
            pltpu.VMEM((_ROWS, 128), jnp.int32),
            pltpu.VMEM((_ROWS, 128), jnp.float32),
            pltpu.VMEM((_ROWS, 128), jnp.float32),
            pltpu.VMEM((_ROWS, 128), jnp.float32),
            pltpu.VMEM((_ROWS, 128), jnp.float32),
            pltpu.VMEM((_W * _W,), jnp.float32),
            pltpu.VMEM((_W * _W,), jnp.float32),
            pltpu.VMEM((_BUF_WORDS,), jnp.float32),
            pltpu.VMEM((_BUF_WORDS,), jnp.float32),
            pltpu.VMEM((_ROWS, 128), jnp.int32),
            pltpu.SemaphoreType.DMA,
            pltpu.SemaphoreType.DMA,
            pltpu.SemaphoreType.DMA,
            pltpu.SemaphoreType.DMA,
            pltpu.SemaphoreType.DMA,
            pltpu.SemaphoreType.DMA,
            pltpu.SemaphoreType.DMA,
            pltpu.SemaphoreType.DMA,
        ],
        compiler_params=pltpu.CompilerParams(use_tc_tiling_on_sc=False,
                                             needs_layout_passes=False),
    )
    return f(idx_full, mask, cube.reshape(-1), dest)


def kernel(cube, to_process, batch_size):
    base, mask = _halo_indices(to_process, batch_size)
    ch_off = (jnp.arange(_C, dtype=jnp.int32) * (_W * _W))[None, :, None]
    idx_full = (base[:, None, :] + ch_off).reshape(_K, _C, _ROWS, 128)
    mask = mask.reshape(_K, _ROWS, 128)
    out = _sc_pad(idx_full, mask, cube, jnp.asarray(_DEST))
    return out.reshape(_K, _C, _WP, _WP)
